# baseline (speedup 1.0000x reference)
import functools

import jax
import jax.numpy as jnp
from jax import lax
from jax.experimental import pallas as pl
from jax.experimental.pallas import tpu as pltpu
from jax.experimental.pallas import tpu_sc as plsc

V, D, B, C = 1000000, 128, 16384, 20
NC, NS, L = 2, 16, 16
NW = NC * NS
BPW = B // NW
CH = 16
NCHUNK = BPW // CH
RPC = CH * C
IW = 80
NP = RPC // IW
TPP = CH // NP
DK = D // L


def _sc_kernel(tgt_idx_hbm, ctx_idx_hbm, tgt_table, ctx_table, out_hbm,
               tidx_v, craw_v, cidx_v, trows, crows, out_v, sems):
  wid = lax.axis_index("s") * NC + lax.axis_index("c")

  lane = lax.iota(jnp.int32, L)
  xor_idx = [lane ^ s for s in (8, 4, 2, 1)]

  pltpu.sync_copy(tgt_idx_hbm.at[pl.ds(wid * BPW, BPW)], tidx_v)
  for c in range(C):
    pltpu.async_copy(ctx_idx_hbm.at[pl.ds(c * B + wid * BPW, BPW)],
                     craw_v.at[c], sems[0][0])
  for c in range(C):
    pltpu.make_async_copy(ctx_idx_hbm.at[pl.ds(c * B + wid * BPW, BPW)],
                          craw_v.at[c], sems[0][0]).wait()

  lane_c = lane * C

  @plsc.parallel_loop(0, BPW // L, unroll=2)
  def repack(w):
    for c in range(C):
      v = craw_v[c, pl.ds(w * L, L)]
      plsc.store_scatter(cidx_v, [lane_c + (w * L * C + c)], v)

  def issue_part(j, q, g):
    if g == 0:
      pltpu.async_copy(tgt_table.at[tidx_v.at[pl.ds(j * CH, CH)]], trows[q],
                       sems[q][0])
    pltpu.async_copy(ctx_table.at[cidx_v.at[pl.ds(j * RPC + g * IW, IW)]],
                     crows[q].at[pl.ds(g * IW, IW)], sems[q][g])

  def drain_part(j, q, g):
    if g == 0:
      pltpu.make_async_copy(tgt_table.at[tidx_v.at[pl.ds(j * CH, CH)]],
                            trows[q], sems[q][0]).wait()
    pltpu.make_async_copy(
        ctx_table.at[cidx_v.at[pl.ds(j * RPC + g * IW, IW)]],
        crows[q].at[pl.ds(g * IW, IW)], sems[q][g]).wait()

  lane_b = lane * BPW

  def compute_group(j, q, g):
    trow_v = trows[q]
    crow_v = crows[q]

    @plsc.parallel_loop(g * TPP, (g + 1) * TPP, unroll=2)
    def tgt_body(i):
      t = [trow_v[i, pl.ds(kk * L, L)] for kk in range(DK)]
      b_local = j * CH + i
      res1 = jnp.zeros((L,), jnp.float32)
      res2 = jnp.zeros((L,), jnp.float32)
      for c in range(C):
        r = i * C + c
        p = [t[kk] * crow_v[r, pl.ds(kk * L, L)] for kk in range(DK)]
        acc = ((p[0] + p[1]) + (p[2] + p[3])) + ((p[4] + p[5]) + (p[6] + p[7]))
        for x in xor_idx:
          acc = acc + acc.at[x].get(mode="promise_in_bounds")
        if c < L:
          res1 = jnp.where(lane == c, acc, res1)
        else:
          res2 = jnp.where(lane == c - L, acc, res2)
      plsc.store_scatter(out_v, [lane_b + b_local], res1)
      plsc.store_scatter(out_v, [lane_b + (L * BPW + b_local)], res2,
                         mask=lane < C - L)

  for g in range(NP):
    issue_part(0, 0, g)

  def chunk_pair(jj, carry):
    j0 = 2 * jj
    j1 = j0 + 1
    for g in range(NP):
      issue_part(j1, 1, g)
      drain_part(j0, 0, g)
      compute_group(j0, 0, g)
    for g in range(NP):
      @pl.when(jj < NCHUNK // 2 - 1)
      def _():
        issue_part(j0 + 2, 0, g)

      drain_part(j1, 1, g)
      compute_group(j1, 1, g)
    return carry

  lax.fori_loop(0, NCHUNK // 2, chunk_pair, 0)

  for c in range(C):
    pltpu.async_copy(out_v.at[pl.ds(c * BPW, BPW)],
                     out_hbm.at[pl.ds(c * B + wid * BPW, BPW)], sems[0][0])
  for c in range(C):
    pltpu.make_async_copy(out_v.at[pl.ds(c * BPW, BPW)],
                          out_hbm.at[pl.ds(c * B + wid * BPW, BPW)],
                          sems[0][0]).wait()


@jax.jit
def _run(tgt_idx, ctx_idx, tgt_table, ctx_table):
  def body(tgt_idx_hbm, ctx_idx_hbm, tgt_table_hbm, ctx_table_hbm, out_hbm,
           tidx_v, craw_v, cidx_v, tr0, cr0, tr1, cr1, out_v,
           s00, s01, s02, s03, s10, s11, s12, s13):
    _sc_kernel(tgt_idx_hbm, ctx_idx_hbm, tgt_table_hbm, ctx_table_hbm,
               out_hbm, tidx_v, craw_v, cidx_v,
               [tr0, tr1], [cr0, cr1], out_v,
               [[s00, s01, s02, s03], [s10, s11, s12, s13]])

  kfn = pl.kernel(
      body,
      out_type=jax.ShapeDtypeStruct((B * C,), jnp.float32),
      mesh=plsc.VectorSubcoreMesh(core_axis_name="c", subcore_axis_name="s"),
      compiler_params=pltpu.CompilerParams(needs_layout_passes=False),
      scratch_types=[
          pltpu.VMEM((BPW,), jnp.int32),
          pltpu.VMEM((C, BPW), jnp.int32),
          pltpu.VMEM((BPW * C,), jnp.int32),
          pltpu.VMEM((CH, D), jnp.float32),
          pltpu.VMEM((RPC, D), jnp.float32),
          pltpu.VMEM((CH, D), jnp.float32),
          pltpu.VMEM((RPC, D), jnp.float32),
          pltpu.VMEM((BPW * C,), jnp.float32),
          pltpu.SemaphoreType.DMA,
          pltpu.SemaphoreType.DMA,
          pltpu.SemaphoreType.DMA,
          pltpu.SemaphoreType.DMA,
          pltpu.SemaphoreType.DMA,
          pltpu.SemaphoreType.DMA,
          pltpu.SemaphoreType.DMA,
          pltpu.SemaphoreType.DMA,
      ],
  )
  return kfn(tgt_idx, ctx_idx, tgt_table, ctx_table)


def kernel(target, context, tgt_table, ctx_table):
  tgt_idx = target.astype(jnp.int32)
  ctx_idx = context.astype(jnp.int32).T.reshape(C * B)
  out = _run(tgt_idx, ctx_idx, tgt_table, ctx_table)
  return out.reshape(C, B).T


# speedup vs baseline: 1.2496x; 1.2496x over previous
import jax
import jax.numpy as jnp
from jax import lax
from jax.experimental import pallas as pl
from jax.experimental.pallas import tpu as pltpu
from jax.experimental.pallas import tpu_sc as plsc

V, D, B, C = 1000000, 128, 16384, 20
NC, NS, L = 2, 16, 16
NW = NC * NS
BPW = B // NW
CH = 16
NCHUNK = BPW // CH
RPC = CH * C
IW = 80
NIR = RPC // IW
DK = D // L


def _sc_kernel(tgt_idx_hbm, ctx_idx_hbm, tgt_table, ctx_table, out_hbm,
               tidx_v, craw_v, cidx_v, trow0_v, crow0_v, trow1_v, crow1_v,
               out_v, sem0, sem1):
  wid = lax.axis_index("s") * NC + lax.axis_index("c")

  lane = lax.iota(jnp.int32, L)
  xor_idx = [lane ^ s for s in (8, 4, 2, 1)]

  pltpu.sync_copy(tgt_idx_hbm.at[pl.ds(wid * BPW, BPW)], tidx_v)
  for c in range(C):
    pltpu.async_copy(ctx_idx_hbm.at[pl.ds(c * B + wid * BPW, BPW)],
                     craw_v.at[c], sem0)
  for c in range(C):
    pltpu.make_async_copy(ctx_idx_hbm.at[pl.ds(c * B + wid * BPW, BPW)],
                          craw_v.at[c], sem0).wait()

  lane_c = lane * C

  @plsc.parallel_loop(0, BPW // L, unroll=2)
  def repack(w):
    for c in range(C):
      v = craw_v[c, pl.ds(w * L, L)]
      plsc.store_scatter(cidx_v, [lane_c + (w * L * C + c)], v)

  def issue(j, trow_v, crow_v, sem):
    pltpu.async_copy(tgt_table.at[tidx_v.at[pl.ds(j * CH, CH)]], trow_v, sem)
    for k in range(NIR):
      pltpu.async_copy(ctx_table.at[cidx_v.at[pl.ds(j * RPC + k * IW, IW)]],
                       crow_v.at[pl.ds(k * IW, IW)], sem)

  def drain(j, trow_v, crow_v, sem):
    pltpu.make_async_copy(tgt_table.at[tidx_v.at[pl.ds(j * CH, CH)]], trow_v,
                          sem).wait()
    for k in range(NIR):
      pltpu.make_async_copy(
          ctx_table.at[cidx_v.at[pl.ds(j * RPC + k * IW, IW)]],
          crow_v.at[pl.ds(k * IW, IW)], sem).wait()

  lane_b = lane * BPW

  def compute(j, trow_v, crow_v):
    @plsc.parallel_loop(0, CH, unroll=2)
    def tgt_body(i):
      t = [trow_v[i, pl.ds(kk * L, L)] for kk in range(DK)]
      b_local = j * CH + i
      res1 = jnp.zeros((L,), jnp.float32)
      res2 = jnp.zeros((L,), jnp.float32)
      for c in range(C):
        r = i * C + c
        p = [t[kk] * crow_v[r, pl.ds(kk * L, L)] for kk in range(DK)]
        acc = ((p[0] + p[1]) + (p[2] + p[3])) + ((p[4] + p[5]) + (p[6] + p[7]))
        for x in xor_idx:
          acc = acc + acc.at[x].get(mode="promise_in_bounds")
        if c < L:
          res1 = jnp.where(lane == c, acc, res1)
        else:
          res2 = jnp.where(lane == c - L, acc, res2)
      plsc.store_scatter(out_v, [lane_b + b_local], res1)
      plsc.store_scatter(out_v, [lane_b + (L * BPW + b_local)], res2,
                         mask=lane < C - L)

  issue(0, trow0_v, crow0_v, sem0)

  def chunk_pair(jj, carry):
    j0 = 2 * jj
    j1 = j0 + 1
    issue(j1, trow1_v, crow1_v, sem1)
    drain(j0, trow0_v, crow0_v, sem0)
    compute(j0, trow0_v, crow0_v)

    @pl.when(jj < NCHUNK // 2 - 1)
    def _():
      issue(j0 + 2, trow0_v, crow0_v, sem0)

    drain(j1, trow1_v, crow1_v, sem1)
    compute(j1, trow1_v, crow1_v)
    return carry

  lax.fori_loop(0, NCHUNK // 2, chunk_pair, 0)

  for c in range(C):
    pltpu.async_copy(out_v.at[pl.ds(c * BPW, BPW)],
                     out_hbm.at[pl.ds(c * B + wid * BPW, BPW)], sem0)
  for c in range(C):
    pltpu.make_async_copy(out_v.at[pl.ds(c * BPW, BPW)],
                          out_hbm.at[pl.ds(c * B + wid * BPW, BPW)],
                          sem0).wait()


@jax.jit
def _run(tgt_idx, ctx_idx, tgt_table, ctx_table):
  kfn = pl.kernel(
      _sc_kernel,
      out_type=jax.ShapeDtypeStruct((B * C,), jnp.float32),
      mesh=plsc.VectorSubcoreMesh(core_axis_name="c", subcore_axis_name="s"),
      compiler_params=pltpu.CompilerParams(needs_layout_passes=False),
      scratch_types=[
          pltpu.VMEM((BPW,), jnp.int32),
          pltpu.VMEM((C, BPW), jnp.int32),
          pltpu.VMEM((BPW * C,), jnp.int32),
          pltpu.VMEM((CH, D), jnp.float32),
          pltpu.VMEM((RPC, D), jnp.float32),
          pltpu.VMEM((CH, D), jnp.float32),
          pltpu.VMEM((RPC, D), jnp.float32),
          pltpu.VMEM((BPW * C,), jnp.float32),
          pltpu.SemaphoreType.DMA,
          pltpu.SemaphoreType.DMA,
      ],
  )
  return kfn(tgt_idx, ctx_idx, tgt_table, ctx_table)


def kernel(target, context, tgt_table, ctx_table):
  tgt_idx = target.astype(jnp.int32)
  ctx_idx = context.astype(jnp.int32).T.reshape(C * B)
  out = _run(tgt_idx, ctx_idx, tgt_table, ctx_table)
  return out.reshape(C, B).T
